# Initial kernel scaffold; baseline (speedup 1.0000x reference)
import jax
import jax.numpy as jnp
from jax.experimental import pallas as pl


def kernel(pair_repr, helix_ends_list, W1, b1, W2, b2):
    raise NotImplementedError("write your pallas kernel here")


# speedup vs baseline: 6.2687x; 6.2687x over previous
import functools
import jax
import jax.numpy as jnp
from jax import lax
from jax.experimental import pallas as pl
from jax.experimental.pallas import tpu as pltpu


def _mlp_body(g1_ref, g2_ref, w1a_ref, w1b_ref, b1_ref, w2_ref, i_ref, out_ref):
    f1 = g1_ref[0, 0, 0]
    f2 = g2_ref[0, 0, 0]
    t = (
        jnp.dot(f1, w1a_ref[...], preferred_element_type=jnp.float32)
        + jnp.dot(f2, w1b_ref[...], preferred_element_type=jnp.float32)
        + b1_ref[...]
    )
    h = jnp.maximum(t, 0.0)
    o = jnp.dot(h, w2_ref[...], preferred_element_type=jnp.float32)
    orow = lax.dot_general(
        o, i_ref[...], (((0,), (0,)), ((), ())),
        preferred_element_type=jnp.float32,
    )
    out_ref[0, 0] = orow


def _mlp_call(gathered, W1a, W1b, b1r, W2, I, B, H, D, interpret=False):
    grid = (B, H)
    return pl.pallas_call(
        _mlp_body,
        grid=grid,
        in_specs=[
            pl.BlockSpec((1, 1, 1, H, D), lambda b, i: (0, b, i, 0, 0)),
            pl.BlockSpec((1, 1, 1, H, D), lambda b, i: (1, b, i, 0, 0)),
            pl.BlockSpec((D, 64), lambda b, i: (0, 0)),
            pl.BlockSpec((D, 64), lambda b, i: (0, 0)),
            pl.BlockSpec((1, 64), lambda b, i: (0, 0)),
            pl.BlockSpec((64, 1), lambda b, i: (0, 0)),
            pl.BlockSpec((H, H), lambda b, i: (0, 0)),
        ],
        out_specs=pl.BlockSpec((1, 1, 1, H), lambda b, i: (b, i, 0, 0)),
        out_shape=jax.ShapeDtypeStruct((B, H, 1, H), jnp.float32),
        interpret=interpret,
    )(gathered, gathered, W1a, W1b, b1r, W2, I)


def kernel(pair_repr, helix_ends_list, W1, b1, W2, b2):
    B, L, _, D = pair_repr.shape
    H = helix_ends_list.shape[1]
    i5 = helix_ends_list[:, :, 1]
    i3 = helix_ends_list[:, :, 2]

    boff = (jnp.arange(B, dtype=jnp.int32) * (L * L))[:, None, None]
    idx1 = boff + i5[:, :, None] * L + i5[:, None, :]
    idx2 = boff + i3[:, :, None] * L + i3[:, None, :]
    idx_all = jnp.stack([idx1, idx2], axis=0)

    table = pair_repr.reshape(B * L * L, D)
    gathered = table[idx_all.reshape(-1)].reshape(2, B, H, H, D)

    W1a = W1[:D]
    W1b = W1[D:]
    I = jnp.eye(H, dtype=jnp.float32)

    out = _mlp_call(gathered, W1a, W1b, b1.reshape(1, 64), W2, I, B, H, D)
    return out.reshape(B, H, H) + b2[0]
